# baseline (speedup 1.0000x reference)
import functools

import jax
import jax.numpy as jnp
from jax import lax
from jax.experimental import pallas as pl
from jax.experimental.pallas import tpu as pltpu
from jax.experimental.pallas import tpu_sc as plsc

N = 10000
D = 128
E = 160000
DE = 16
DM = 4
DOUT = 32

NC = 2
NS = 16
NW = NC * NS

CH = 128
RING = 20
EP = 163840
EPW = EP // NW
NCHUNK = EPW // CH
NPAD = 10008
YW = 128
AW = 16
NAGG = 10240
ROWS_PER_TILE = NAGG // NS


def _prep_body(xo_ref, xi_ref, w0_ref, w1_ref, y0_ref, y1_ref):
    y0_ref[:N, :] = jnp.dot(xo_ref[...], w0_ref[...],
                            preferred_element_type=jnp.float32)
    y0_ref[N:, :] = jnp.zeros((NPAD - N, YW), jnp.float32)
    y1_ref[:N, :] = jnp.dot(xi_ref[...], w1_ref[...],
                            preferred_element_type=jnp.float32)
    y1_ref[N:, :] = jnp.zeros((NPAD - N, YW), jnp.float32)


def _epilogue_body(a0_ref, a1_ref, xi_ref, rw0_ref, rw1_ref, b0_ref, b1_ref,
                   la_ref, lb_ref, lbias_ref, out_ref):
    xi = xi_ref[...]
    s0 = jax.nn.sigmoid(a0_ref[0, :N, :] + a0_ref[1, :N, :]
                        + jnp.dot(xi, rw0_ref[...],
                                  preferred_element_type=jnp.float32)
                        + b0_ref[...])
    s1 = jax.nn.sigmoid(a1_ref[0, :N, :] + a1_ref[1, :N, :]
                        + jnp.dot(xi, rw1_ref[...],
                                  preferred_element_type=jnp.float32)
                        + b1_ref[...])
    out_ref[...] = jax.nn.sigmoid(
        jnp.dot(s0, la_ref[...], preferred_element_type=jnp.float32)
        + jnp.dot(s1, lb_ref[...], preferred_element_type=jnp.float32)
        + lbias_ref[...])


def _edge_body(y0, y1, s0, d0, a0, s1, d1, a1, out0, out1,
               agg0, agg1, srcall, dstall, eab0, eab1, ygb0, ygb1, mball,
               semg0, semg1, seme0, seme1, semsc):
    cid = lax.axis_index("c")
    sid = lax.axis_index("s")
    eabs = (eab0, eab1)
    ygbs = (ygb0, ygb1)
    semgs = (semg0, semg1)
    semes = (seme0, seme1)

    zero16 = jnp.zeros((16,), jnp.float32)

    def _zloop(i, c):
        mball[i, :] = zero16
        return c
    lax.fori_loop(0, ROWS_PER_TILE, _zloop, 0)
    zsl = pl.ds(0, ROWS_PER_TILE)
    asl = pl.ds(sid * ROWS_PER_TILE, ROWS_PER_TILE)
    pltpu.sync_copy(mball.at[zsl, :], agg0.at[asl, :])
    pltpu.sync_copy(mball.at[zsl, :], agg1.at[asl, :])
    plsc.subcore_barrier()

    wid = cid * NS + sid
    lanes = lax.iota(jnp.int32, 16)
    ea_idx = [(lanes >> 2) + 4 * j for j in range(4)]
    rot8 = (lanes + 8) & 15
    rot4 = (lanes + 4) & 15
    low4 = lanes < DM

    def _take(v, idx):
        return lax.gather(
            v, idx[:, None],
            lax.GatherDimensionNumbers(offset_dims=(),
                                       collapsed_slice_dims=(0,),
                                       start_index_map=(0,)),
            slice_sizes=(1,),
            mode=lax.GatherScatterMode.PROMISE_IN_BOUNDS)

    def _run_step(yt, st, dt, at, aggt):
        pltpu.sync_copy(st.at[wid], srcall)
        pltpu.sync_copy(dt.at[wid], dstall)

        def _ce(c):
            return jnp.minimum(wid * NCHUNK + c, E // CH - 1)

        def _fetch(c, b):
            pltpu.async_copy(yt.at[srcall.at[pl.ds(c * CH, CH)]],
                             ygbs[b], semgs[b])
            pltpu.async_copy(at.at[_ce(c)], eabs[b], semes[b])

        def _fetch_wait(c, b):
            pltpu.make_async_copy(yt.at[srcall.at[pl.ds(c * CH, CH)]],
                                  ygbs[b], semgs[b]).wait()
            pltpu.make_async_copy(at.at[_ce(c)], eabs[b], semes[b]).wait()

        def _compute(c, b, s):
            eab = eabs[b]
            ygb = ygbs[b]

            def _edge4(i, cc):
                for u in range(4):
                    el = i * 4 + u
                    ea_v = eab[el, :]
                    yb_v = ygb[el, pl.ds(64, 16)]
                    acc = ygb[el, pl.ds(0, 16)] * _take(ea_v, ea_idx[0])
                    for j in range(1, 4):
                        acc = acc + ygb[el, pl.ds(16 * j, 16)] * _take(ea_v, ea_idx[j])
                    acc = acc + _take(acc, rot8)
                    acc = acc + _take(acc, rot4)
                    mball[s * CH + el, :] = jnp.where(low4, acc, 0.0) + yb_v
                return cc
            lax.fori_loop(0, CH // 4, _edge4, 0)

        def _scatter(c, s):
            pltpu.async_copy(mball.at[pl.ds(s * CH, CH), :],
                             aggt.at[dstall.at[pl.ds(c * CH, CH)]],
                             semsc, add=True)

        def _drain(c, s):
            pltpu.make_async_copy(mball.at[pl.ds(s * CH, CH), :],
                                  aggt.at[dstall.at[pl.ds(c * CH, CH)]],
                                  semsc).wait()

        _fetch(0, 0)
        _fetch(1, 1)

        def _pair1(p, carry):
            for b in range(2):
                c = 2 * p + b
                _fetch_wait(c, b)
                _compute(c, b, c)
                _scatter(c, c)
                _fetch(c + 2, b)
            return carry
        lax.fori_loop(0, RING // 2, _pair1, 0)

        def _pair2(p, carry):
            for b in range(2):
                c = RING + 2 * p + b
                s = 2 * p + b
                _drain(s, s)
                _fetch_wait(c, b)
                _compute(c, b, s)
                _scatter(c, s)
                _fetch(c + 2, b)
            return carry
        lax.fori_loop(0, (NCHUNK - RING) // 2 - 1, _pair2, 0)
        for b in range(2):
            c = NCHUNK - 2 + b
            s = c - RING
            _drain(s, s)
            _fetch_wait(c, b)
            _compute(c, b, s)
            _scatter(c, s)

        def _drain2(s, carry):
            _drain(RING + s, s)
            return carry
        lax.fori_loop(0, RING, _drain2, 0)

    _run_step(y0, s0, d0, a0, agg0)
    _run_step(y1, s1, d1, a1, agg1)
    plsc.subcore_barrier()

    sl = pl.ds(sid * ROWS_PER_TILE, ROWS_PER_TILE)
    pltpu.sync_copy(agg0.at[sl, :], out0.at[cid, sl, :])
    pltpu.sync_copy(agg1.at[sl, :], out1.at[cid, sl, :])


@jax.jit
def _impl(x_indivi, x_other, edge_index_oi, edge_attr_oi, edge_index_ii,
          edge_attr_ii, nn_w0, nn_b0, root_w0, bias0, nn_w1, nn_b1, root_w1,
          bias1, lin_w, lin_b):
    f32 = jnp.float32

    def _wcat(nn_w, nn_b):
        wr = nn_w.reshape(D, DM, DE).transpose(0, 2, 1).reshape(D, DM * DE)
        br = nn_b.reshape(D, DM)
        return jnp.concatenate(
            [wr, br, jnp.zeros((D, YW - DM * DE - DM), f32)], axis=1)

    w0 = _wcat(nn_w0, nn_b0)
    w1 = _wcat(nn_w1, nn_b1)

    npad = EP - E
    def _padded(ei, ea):
        src = jnp.concatenate([ei[0], jnp.full((npad,), N, jnp.int32)])
        dst = jnp.concatenate([ei[1], jnp.zeros((npad,), jnp.int32)])
        return (src.reshape(NW, EPW),
                dst.reshape(NW, EPW),
                ea.reshape(E // CH, CH, DE))

    s0, d0, a0 = _padded(edge_index_oi, edge_attr_oi)
    s1, d1, a1 = _padded(edge_index_ii, edge_attr_ii)

    y0, y1 = pl.pallas_call(
        _prep_body,
        out_shape=(jax.ShapeDtypeStruct((NPAD, YW), f32),
                   jax.ShapeDtypeStruct((NPAD, YW), f32)),
    )(x_other, x_indivi, w0, w1)

    mesh = plsc.VectorSubcoreMesh(core_axis_name="c", subcore_axis_name="s",
                                  num_cores=NC, num_subcores=NS)
    agg0, agg1 = pl.kernel(
        _edge_body,
        out_type=(jax.ShapeDtypeStruct((NC, NAGG, AW), f32),
                  jax.ShapeDtypeStruct((NC, NAGG, AW), f32)),
        mesh=mesh,
        compiler_params=pltpu.CompilerParams(use_tc_tiling_on_sc=False),
        scratch_types=[
            pltpu.VMEM_SHARED((NAGG, AW), f32),
            pltpu.VMEM_SHARED((NAGG, AW), f32),
            pltpu.VMEM((EPW,), jnp.int32),
            pltpu.VMEM((EPW,), jnp.int32),
            pltpu.VMEM((CH, DE), f32),
            pltpu.VMEM((CH, DE), f32),
            pltpu.VMEM((CH, YW), f32),
            pltpu.VMEM((CH, YW), f32),
            pltpu.VMEM((RING * CH, AW), f32),
            pltpu.SemaphoreType.DMA,
            pltpu.SemaphoreType.DMA,
            pltpu.SemaphoreType.DMA,
            pltpu.SemaphoreType.DMA,
            pltpu.SemaphoreType.DMA,
        ],
    )(y0, y1, s0, d0, a0, s1, d1, a1)

    def _rootpad(root_w, bias):
        rw = jnp.zeros((D, AW), f32).at[:, :DM].set(root_w.T)
        b = jnp.zeros((1, AW), f32).at[0, :DM].set(bias)
        return rw, b

    rw0, b0 = _rootpad(root_w0, bias0)
    rw1, b1 = _rootpad(root_w1, bias1)
    la = jnp.zeros((AW, DOUT), f32).at[:DM, :].set(lin_w[:, :DM].T)
    lb = jnp.zeros((AW, DOUT), f32).at[:DM, :].set(lin_w[:, DM:].T)
    lbias = lin_b.reshape(1, DOUT)

    return pl.pallas_call(
        _epilogue_body,
        out_shape=jax.ShapeDtypeStruct((N, DOUT), f32),
    )(agg0, agg1, x_indivi, rw0, rw1, b0, b1, la, lb, lbias)


def kernel(x_indivi, x_other, edge_index_oi, edge_attr_oi, edge_index_ii,
           edge_attr_ii, nn_w0, nn_b0, root_w0, bias0, nn_w1, nn_b1, root_w1,
           bias1, lin_w, lin_b):
    return _impl(x_indivi, x_other, edge_index_oi, edge_attr_oi, edge_index_ii,
                 edge_attr_ii, nn_w0, nn_b0, root_w0, bias0, nn_w1, nn_b1,
                 root_w1, bias1, lin_w, lin_b)


# speedup vs baseline: 1.3373x; 1.3373x over previous
import functools

import jax
import jax.numpy as jnp
from jax import lax
from jax.experimental import pallas as pl
from jax.experimental.pallas import tpu as pltpu
from jax.experimental.pallas import tpu_sc as plsc

N = 10000
D = 128
E = 160000
DE = 16
DM = 4
DOUT = 32

NC = 2
NS = 16
NW = NC * NS

CH = 128
RING = 20
EP = 163840
EPW = EP // NW
NCHUNK = EPW // CH
NPAD = 10008
YW = 80
NCH_C0 = 50
NCH_C1 = 30
NCH_MAX = max(NCH_C0, NCH_C1)
EPW_MAX = NCH_MAX * CH
PADE = max(EP, (NS * NCH_C0 + (NS - 1) * NCH_C1) * CH + EPW_MAX,
           (NS * NCH_C1 + (NS - 1) * NCH_C0) * CH + EPW_MAX)
AW = 16
NAGG = 10240
ROWS_PER_TILE = NAGG // NS


def _prep_body(xo_ref, xi_ref, w0_ref, w1_ref, y0_ref, y1_ref):
    y0_ref[:N, :] = jnp.dot(xo_ref[...], w0_ref[...],
                            preferred_element_type=jnp.float32)
    y0_ref[N:, :] = jnp.zeros((NPAD - N, YW), jnp.float32)
    y1_ref[:N, :] = jnp.dot(xi_ref[...], w1_ref[...],
                            preferred_element_type=jnp.float32)
    y1_ref[N:, :] = jnp.zeros((NPAD - N, YW), jnp.float32)


def _epilogue_body(a0_ref, a1_ref, xi_ref, rw0_ref, rw1_ref, b0_ref, b1_ref,
                   la_ref, lb_ref, lbias_ref, out_ref):
    xi = xi_ref[...]
    s0 = jax.nn.sigmoid(a0_ref[0, :N, :] + a0_ref[1, :N, :]
                        + jnp.dot(xi, rw0_ref[...],
                                  preferred_element_type=jnp.float32)
                        + b0_ref[...])
    s1 = jax.nn.sigmoid(a1_ref[0, :N, :] + a1_ref[1, :N, :]
                        + jnp.dot(xi, rw1_ref[...],
                                  preferred_element_type=jnp.float32)
                        + b1_ref[...])
    out_ref[...] = jax.nn.sigmoid(
        jnp.dot(s0, la_ref[...], preferred_element_type=jnp.float32)
        + jnp.dot(s1, lb_ref[...], preferred_element_type=jnp.float32)
        + lbias_ref[...])


def _edge_body(y0, y1, s0, d0, a0, s1, d1, a1, out0, out1,
               agg0, agg1, srcall, dstall, eab0, eab1, ygb0, ygb1, mball,
               semg0, semg1, seme0, seme1, semsc):
    cid = lax.axis_index("c")
    sid = lax.axis_index("s")
    eabs = (eab0, eab1)
    ygbs = (ygb0, ygb1)
    semgs = (semg0, semg1)
    semes = (seme0, seme1)

    zero16 = jnp.zeros((16,), jnp.float32)

    def _zloop(i, c):
        mball[i, :] = zero16
        return c
    lax.fori_loop(0, ROWS_PER_TILE, _zloop, 0)
    zsl = pl.ds(0, ROWS_PER_TILE)
    asl = pl.ds(sid * ROWS_PER_TILE, ROWS_PER_TILE)
    pltpu.sync_copy(mball.at[zsl, :], agg0.at[asl, :])
    pltpu.sync_copy(mball.at[zsl, :], agg1.at[asl, :])
    plsc.subcore_barrier()

    ncw = NCH_C0 + cid * (NCH_C1 - NCH_C0)
    chunkbase = cid * NS * NCH_C0 + sid * ncw
    lanes = lax.iota(jnp.int32, 16)
    ea_idx = [(lanes >> 2) + 4 * j for j in range(4)]
    rot8 = (lanes + 8) & 15
    rot4 = (lanes + 4) & 15
    low4 = lanes < DM

    def _take(v, idx):
        return lax.gather(
            v, idx[:, None],
            lax.GatherDimensionNumbers(offset_dims=(),
                                       collapsed_slice_dims=(0,),
                                       start_index_map=(0,)),
            slice_sizes=(1,),
            mode=lax.GatherScatterMode.PROMISE_IN_BOUNDS)

    def _run_step(yt, st, dt, at, aggt):
        ebase = chunkbase * CH
        pltpu.sync_copy(st.at[pl.ds(ebase, EPW_MAX)], srcall)
        pltpu.sync_copy(dt.at[pl.ds(ebase, EPW_MAX)], dstall)

        def _ce(c):
            return jnp.minimum(chunkbase + c, E // CH - 1)

        def _fetch(c, b):
            pltpu.async_copy(yt.at[srcall.at[pl.ds(c * CH, CH)]],
                             ygbs[b], semgs[b])
            pltpu.async_copy(at.at[_ce(c)], eabs[b], semes[b])

        def _fetch_wait(c, b):
            pltpu.make_async_copy(yt.at[srcall.at[pl.ds(c * CH, CH)]],
                                  ygbs[b], semgs[b]).wait()
            pltpu.make_async_copy(at.at[_ce(c)], eabs[b], semes[b]).wait()

        def _compute(c, b, s):
            eab = eabs[b]
            ygb = ygbs[b]

            def _edge4(i, cc):
                for u in range(4):
                    el = i * 4 + u
                    ea_v = eab[el, :]
                    yb_v = ygb[el, pl.ds(64, 16)]
                    acc = ygb[el, pl.ds(0, 16)] * _take(ea_v, ea_idx[0])
                    for j in range(1, 4):
                        acc = acc + ygb[el, pl.ds(16 * j, 16)] * _take(ea_v, ea_idx[j])
                    acc = acc + _take(acc, rot8)
                    acc = acc + _take(acc, rot4)
                    mball[s * CH + el, :] = jnp.where(low4, acc, 0.0) + yb_v
                return cc
            lax.fori_loop(0, CH // 4, _edge4, 0)

        def _scatter(c, s):
            pltpu.async_copy(mball.at[pl.ds(s * CH, CH), :],
                             aggt.at[dstall.at[pl.ds(c * CH, CH)]],
                             semsc, add=True)

        def _drain(c, s):
            pltpu.make_async_copy(mball.at[pl.ds(s * CH, CH), :],
                                  aggt.at[dstall.at[pl.ds(c * CH, CH)]],
                                  semsc).wait()

        _fetch(0, 0)
        _fetch(1, 1)

        def _pair1(p, carry):
            for b in range(2):
                c = 2 * p + b
                _fetch_wait(c, b)
                _compute(c, b, c)
                _scatter(c, c)
                _fetch(c + 2, b)
            return carry
        lax.fori_loop(0, RING // 2, _pair1, 0)

        def _pair2(p, carry):
            for b in range(2):
                c = RING + 2 * p + b
                s = lax.rem(c, RING)
                _drain(c - RING, s)
                _fetch_wait(c, b)
                _compute(c, b, s)
                _scatter(c, s)
                _fetch(c + 2, b)
            return carry
        lax.fori_loop(0, (ncw - RING) // 2 - 1, _pair2, 0)
        for b in range(2):
            c = ncw - 2 + b
            s = lax.rem(c, RING)
            _drain(c - RING, s)
            _fetch_wait(c, b)
            _compute(c, b, s)
            _scatter(c, s)

        def _drain2(i, carry):
            c = ncw - RING + i
            _drain(c, lax.rem(c, RING))
            return carry
        lax.fori_loop(0, RING, _drain2, 0)

    _run_step(y0, s0, d0, a0, agg0)
    _run_step(y1, s1, d1, a1, agg1)
    plsc.subcore_barrier()

    sl = pl.ds(sid * ROWS_PER_TILE, ROWS_PER_TILE)
    pltpu.sync_copy(agg0.at[sl, :], out0.at[cid, sl, :])
    pltpu.sync_copy(agg1.at[sl, :], out1.at[cid, sl, :])


@jax.jit
def _impl(x_indivi, x_other, edge_index_oi, edge_attr_oi, edge_index_ii,
          edge_attr_ii, nn_w0, nn_b0, root_w0, bias0, nn_w1, nn_b1, root_w1,
          bias1, lin_w, lin_b):
    f32 = jnp.float32

    def _wcat(nn_w, nn_b):
        wr = nn_w.reshape(D, DM, DE).transpose(0, 2, 1).reshape(D, DM * DE)
        br = nn_b.reshape(D, DM)
        return jnp.concatenate(
            [wr, br, jnp.zeros((D, YW - DM * DE - DM), f32)], axis=1)

    w0 = _wcat(nn_w0, nn_b0)
    w1 = _wcat(nn_w1, nn_b1)

    npad = PADE - E
    def _padded(ei, ea):
        src = jnp.concatenate([ei[0], jnp.full((npad,), N, jnp.int32)])
        dst = jnp.concatenate([ei[1], jnp.zeros((npad,), jnp.int32)])
        return src, dst, ea.reshape(E // CH, CH, DE)

    s0, d0, a0 = _padded(edge_index_oi, edge_attr_oi)
    s1, d1, a1 = _padded(edge_index_ii, edge_attr_ii)

    y0, y1 = pl.pallas_call(
        _prep_body,
        out_shape=(jax.ShapeDtypeStruct((NPAD, YW), f32),
                   jax.ShapeDtypeStruct((NPAD, YW), f32)),
    )(x_other, x_indivi, w0, w1)

    mesh = plsc.VectorSubcoreMesh(core_axis_name="c", subcore_axis_name="s",
                                  num_cores=NC, num_subcores=NS)
    agg0, agg1 = pl.kernel(
        _edge_body,
        out_type=(jax.ShapeDtypeStruct((NC, NAGG, AW), f32),
                  jax.ShapeDtypeStruct((NC, NAGG, AW), f32)),
        mesh=mesh,
        compiler_params=pltpu.CompilerParams(use_tc_tiling_on_sc=False),
        scratch_types=[
            pltpu.VMEM_SHARED((NAGG, AW), f32),
            pltpu.VMEM_SHARED((NAGG, AW), f32),
            pltpu.VMEM((EPW_MAX,), jnp.int32),
            pltpu.VMEM((EPW_MAX,), jnp.int32),
            pltpu.VMEM((CH, DE), f32),
            pltpu.VMEM((CH, DE), f32),
            pltpu.VMEM((CH, YW), f32),
            pltpu.VMEM((CH, YW), f32),
            pltpu.VMEM((RING * CH, AW), f32),
            pltpu.SemaphoreType.DMA,
            pltpu.SemaphoreType.DMA,
            pltpu.SemaphoreType.DMA,
            pltpu.SemaphoreType.DMA,
            pltpu.SemaphoreType.DMA,
        ],
    )(y0, y1, s0, d0, a0, s1, d1, a1)

    def _rootpad(root_w, bias):
        rw = jnp.zeros((D, AW), f32).at[:, :DM].set(root_w.T)
        b = jnp.zeros((1, AW), f32).at[0, :DM].set(bias)
        return rw, b

    rw0, b0 = _rootpad(root_w0, bias0)
    rw1, b1 = _rootpad(root_w1, bias1)
    la = jnp.zeros((AW, DOUT), f32).at[:DM, :].set(lin_w[:, :DM].T)
    lb = jnp.zeros((AW, DOUT), f32).at[:DM, :].set(lin_w[:, DM:].T)
    lbias = lin_b.reshape(1, DOUT)

    return pl.pallas_call(
        _epilogue_body,
        out_shape=jax.ShapeDtypeStruct((N, DOUT), f32),
    )(agg0, agg1, x_indivi, rw0, rw1, b0, b1, la, lb, lbias)


def kernel(x_indivi, x_other, edge_index_oi, edge_attr_oi, edge_index_ii,
           edge_attr_ii, nn_w0, nn_b0, root_w0, bias0, nn_w1, nn_b1, root_w1,
           bias1, lin_w, lin_b):
    return _impl(x_indivi, x_other, edge_index_oi, edge_attr_oi, edge_index_ii,
                 edge_attr_ii, nn_w0, nn_b0, root_w0, bias0, nn_w1, nn_b1,
                 root_w1, bias1, lin_w, lin_b)
